# baseline (speedup 1.0000x reference)
import functools

import jax
import jax.numpy as jnp
from jax.experimental import pallas as pl

_SCALING = 32.0 / 16.0
_NT = (((1,), (1,)), ((), ()))


def _ffn_body(E, R, logits_ref, x_ref, w1_ref, w3_ref, w2_ref,
              a1_ref, b1_ref, a3_ref, b3_ref, a2_ref, b2_ref,
              out_ref):
    f32 = jnp.float32
    bf16 = jnp.bfloat16
    dg = jax.lax.dot_general
    xb = x_ref[...].astype(bf16)
    logits = logits_ref[...]
    m = jnp.max(logits, axis=-1, keepdims=True)
    ids_e = jax.lax.broadcasted_iota(jnp.int32, logits.shape, 1)
    e = jnp.min(jnp.where(logits == m, ids_e, E), axis=-1, keepdims=True)
    ids = jax.lax.broadcasted_iota(jnp.int32, (xb.shape[0], E * R), 1)
    mask = (ids // R == e).astype(f32)

    la1 = (dg(xb, a1_ref[...], _NT, preferred_element_type=f32) * mask).astype(bf16)
    g = (dg(xb, w1_ref[...], _NT, preferred_element_type=f32)
         + jnp.dot(la1, b1_ref[...], preferred_element_type=f32))
    la3 = (dg(xb, a3_ref[...], _NT, preferred_element_type=f32) * mask).astype(bf16)
    u = (dg(xb, w3_ref[...], _NT, preferred_element_type=f32)
         + jnp.dot(la3, b3_ref[...], preferred_element_type=f32))
    h = ((g * jax.lax.logistic(g)) * u).astype(bf16)
    la2 = (dg(h, a2_ref[...], _NT, preferred_element_type=f32) * mask).astype(bf16)
    out_ref[...] = (dg(h, w2_ref[...], _NT, preferred_element_type=f32)
                    + jnp.dot(la2, b2_ref[...], preferred_element_type=f32))


def kernel(data, gate_weight, w1, w2, w3,
           lora_a1, lora_b1, lora_a3, lora_b3, lora_a2, lora_b2):
    T, D = data.shape
    F = w1.shape[0]
    E, R, _ = lora_a1.shape
    s = _SCALING
    bf16 = jnp.bfloat16

    router_logits = data @ gate_weight.T

    w1b, w3b = w1.astype(bf16), w3.astype(bf16)
    w2b = w2.astype(bf16)
    a1b = lora_a1.reshape(E * R, D).astype(bf16)
    b1c = (lora_b1.transpose(0, 2, 1).reshape(E * R, F) * s).astype(bf16)
    a3b = lora_a3.reshape(E * R, D).astype(bf16)
    b3c = (lora_b3.transpose(0, 2, 1).reshape(E * R, F) * s).astype(bf16)
    a2b = lora_a2.reshape(E * R, F).astype(bf16)
    b2c = (lora_b2.transpose(0, 2, 1).reshape(E * R, D) * s).astype(bf16)

    TB = 256
    grid = (T // TB,)
    tok = lambda i: (i, 0)
    rep = lambda i: (0, 0)

    out = pl.pallas_call(
        functools.partial(_ffn_body, E, R),
        grid=grid,
        in_specs=[
            pl.BlockSpec((TB, E), tok),
            pl.BlockSpec((TB, D), tok),
            pl.BlockSpec((F, D), rep),
            pl.BlockSpec((F, D), rep),
            pl.BlockSpec((D, F), rep),
            pl.BlockSpec((E * R, D), rep),
            pl.BlockSpec((E * R, F), rep),
            pl.BlockSpec((E * R, D), rep),
            pl.BlockSpec((E * R, F), rep),
            pl.BlockSpec((E * R, F), rep),
            pl.BlockSpec((E * R, D), rep),
        ],
        out_specs=pl.BlockSpec((TB, D), tok),
        out_shape=jax.ShapeDtypeStruct((T, D), data.dtype),
    )(router_logits, data, w1b, w3b, w2b, a1b, b1c, a3b, b3c, a2b, b2c)
    return out, router_logits
